# baseline (speedup 1.0000x reference)
import jax
import jax.numpy as jnp
from jax import lax
from jax.experimental import pallas as pl
from jax.experimental.pallas import tpu as pltpu
from jax.experimental.pallas import tpu_sc as plsc

_NUM_CORES = 2
_NUM_SUBCORES = 16
_NW = _NUM_CORES * _NUM_SUBCORES
_B, _V = 4096, 102
_D = 64
_VOCAB = 102
_TOTAL = _B * _V
_PER_W = _TOTAL // _NW
_BLK_ROWS = 384
_NBLK = _PER_W // _BLK_ROWS
_GRP = 16
_NGRP = _BLK_ROWS // _GRP
_L = 16


def _sc_body(idx_hbm, table_hbm, out_hbm, idx_v, table_v, bufs, ssems):
    wid = lax.axis_index("s") * _NUM_CORES + lax.axis_index("c")
    row_base = wid * _PER_W

    pltpu.sync_copy(idx_hbm.at[pl.ds(wid * (_B // _NW), _B // _NW)], idx_v)
    pltpu.sync_copy(table_hbm, table_v)

    iota = lax.iota(jnp.int32, _L)
    iota_d = iota * _D

    def compute_block(blk, buf):
        def grp(g, carry):
            o = blk * _BLK_ROWS + g * _GRP
            lanes = o + iota
            iv = plsc.load_gather(idx_v, [lanes // _V, lanes % _V])
            tbase = iv * _D
            soff = g * (_GRP * _D) + iota_d
            for d in range(_D):
                col = plsc.load_gather(table_v, [tbase + d])
                plsc.store_scatter(buf, [soff + d], col)
            return carry

        lax.fori_loop(0, _NGRP, grp, 0)

    def out_slice(blk):
        start = (row_base + blk * _BLK_ROWS) * _D
        return out_hbm.at[pl.ds(start, _BLK_ROWS * _D)]

    def store(blk, b):
        pltpu.async_copy(bufs[b], out_slice(blk), ssems[b])

    def wait_store(blk, b):
        pltpu.make_async_copy(bufs[b], out_slice(blk), ssems[b]).wait()

    for b in (0, 1):
        compute_block(b, bufs[b])
        store(b, b)

    def body(p, carry):
        for b in (0, 1):
            blk = p * 2 + b
            wait_store(blk - 2, b)
            compute_block(blk, bufs[b])
            store(blk, b)
        return carry

    lax.fori_loop(1, _NBLK // 2, body, 0)

    wait_store(_NBLK - 2, 0)
    wait_store(_NBLK - 1, 1)


@jax.jit
def _lookup(indices, table_flat):
    mesh = plsc.VectorSubcoreMesh(core_axis_name="c", subcore_axis_name="s")
    f = pl.kernel(
        _sc_body,
        out_type=jax.ShapeDtypeStruct((_TOTAL * _D,), jnp.float32),
        mesh=mesh,
        scratch_types=[
            pltpu.VMEM((_B // _NW, _V), jnp.int32),
            pltpu.VMEM((_VOCAB * _D,), jnp.float32),
            [pltpu.VMEM((_BLK_ROWS * _D,), jnp.float32) for _ in range(2)],
            [pltpu.SemaphoreType.DMA for _ in range(2)],
        ],
        compiler_params=pltpu.CompilerParams(use_tc_tiling_on_sc=False, needs_layout_passes=False),
    )
    return f(indices, table_flat)


def kernel(indices, table):
    out = _lookup(indices, table.reshape(_VOCAB * _D))
    return out.reshape(_B, _V, _D)


# speedup vs baseline: 2.5948x; 2.5948x over previous
import jax
import jax.numpy as jnp
from jax import lax
from jax.experimental import pallas as pl
from jax.experimental.pallas import tpu as pltpu
from jax.experimental.pallas import tpu_sc as plsc

_NUM_CORES = 2
_NUM_SUBCORES = 16
_NW = _NUM_CORES * _NUM_SUBCORES
_B, _V = 4096, 102
_D = 64
_BPW = _B // _NW
_L = 16
_NJB = _BPW // _L


def _sc_body(idx_hbm, table_hbm, out_hbm, idx_v, table_v, bufs, ssems):
    wid = lax.axis_index("s") * _NUM_CORES + lax.axis_index("c")
    bcol = wid * _BPW

    pltpu.sync_copy(idx_hbm.at[pl.ds(bcol, _BPW)], idx_v)
    pltpu.sync_copy(table_hbm, table_v)

    iota = lax.iota(jnp.int32, _L)

    def compute_block(v, buf):
        for jb in range(_NJB):
            lanes = jb * _L + iota
            iv = plsc.load_gather(idx_v, [lanes, v + (iota - iota)])
            tb = iv * _D

            @plsc.parallel_loop(0, _D, step=1, unroll=16)
            def dloop(d):
                col = plsc.load_gather(table_v, [tb + d])
                buf[d, pl.ds(jb * _L, _L)] = col

    def out_slice(v):
        return out_hbm.at[v, :, pl.ds(bcol, _BPW)]

    def store(v, b):
        pltpu.async_copy(bufs[b], out_slice(v), ssems[b])

    def wait_store(v, b):
        pltpu.make_async_copy(bufs[b], out_slice(v), ssems[b]).wait()

    for b in (0, 1):
        compute_block(jnp.int32(b), bufs[b])
        store(b, b)

    def body(p, carry):
        for b in (0, 1):
            v = p * 2 + b
            wait_store(v - 2, b)
            compute_block(v, bufs[b])
            store(v, b)
        return carry

    lax.fori_loop(1, _V // 2, body, 0)

    wait_store(_V - 2, 0)
    wait_store(_V - 1, 1)


@jax.jit
def _lookup(indices, table_flat):
    mesh = plsc.VectorSubcoreMesh(core_axis_name="c", subcore_axis_name="s")
    f = pl.kernel(
        _sc_body,
        out_type=jax.ShapeDtypeStruct((_V, _D, _B), jnp.float32),
        mesh=mesh,
        scratch_types=[
            pltpu.VMEM((_BPW, _V), jnp.int32),
            pltpu.VMEM((_V * _D,), jnp.float32),
            [pltpu.VMEM((_D, _BPW), jnp.float32) for _ in range(2)],
            [pltpu.SemaphoreType.DMA for _ in range(2)],
        ],
        compiler_params=pltpu.CompilerParams(
            use_tc_tiling_on_sc=False, needs_layout_passes=False
        ),
    )
    return f(indices, table_flat)


def kernel(indices, table):
    out_t = _lookup(indices, table.reshape(_V * _D))
    return out_t.transpose(2, 0, 1)


# speedup vs baseline: 6.7104x; 2.5861x over previous
import jax
import jax.numpy as jnp
from jax import lax
from jax.experimental import pallas as pl
from jax.experimental.pallas import tpu as pltpu
from jax.experimental.pallas import tpu_sc as plsc

_NUM_CORES = 2
_NUM_SUBCORES = 16
_NW = _NUM_CORES * _NUM_SUBCORES
_B, _V = 4096, 102
_D = 64
_BPW = _B // _NW
_L = 16
_NJB = _BPW // _L


def _sc_body(idx_hbm, table_hbm, out_hbm, idx_v, table_v, bufs, ssems):
    wid = lax.axis_index("s") * _NUM_CORES + lax.axis_index("c")
    bcol = wid * _BPW

    pltpu.sync_copy(idx_hbm.at[pl.ds(bcol, _BPW)], idx_v)
    pltpu.sync_copy(table_hbm, table_v)

    iota = lax.iota(jnp.int32, _L)

    def compute_block(v, buf):
        for jb in range(_NJB):
            lanes = jb * _L + iota
            iv = plsc.load_gather(idx_v, [lanes, v + (iota - iota)])

            @plsc.parallel_loop(0, _D, step=1, unroll=16)
            def dloop(d):
                col = plsc.load_gather(table_v, [iv + d * _V])
                buf[d, pl.ds(jb * _L, _L)] = col

    def out_slice(v):
        return out_hbm.at[v, :, pl.ds(bcol, _BPW)]

    def store(v, b):
        pltpu.async_copy(bufs[b], out_slice(v), ssems[b])

    def wait_store(v, b):
        pltpu.make_async_copy(bufs[b], out_slice(v), ssems[b]).wait()

    for b in (0, 1):
        compute_block(jnp.int32(b), bufs[b])
        store(b, b)

    def body(p, carry):
        for b in (0, 1):
            v = p * 2 + b
            wait_store(v - 2, b)
            compute_block(v, bufs[b])
            store(v, b)
        return carry

    lax.fori_loop(1, _V // 2, body, 0)

    wait_store(_V - 2, 0)
    wait_store(_V - 1, 1)


@jax.jit
def _lookup(indices, table_flat):
    mesh = plsc.VectorSubcoreMesh(core_axis_name="c", subcore_axis_name="s")
    f = pl.kernel(
        _sc_body,
        out_type=jax.ShapeDtypeStruct((_V, _D, _B), jnp.float32),
        mesh=mesh,
        scratch_types=[
            pltpu.VMEM((_BPW, _V), jnp.int32),
            pltpu.VMEM((_V * _D,), jnp.float32),
            [pltpu.VMEM((_D, _BPW), jnp.float32) for _ in range(2)],
            [pltpu.SemaphoreType.DMA for _ in range(2)],
        ],
        compiler_params=pltpu.CompilerParams(
            use_tc_tiling_on_sc=False, needs_layout_passes=False
        ),
    )
    return f(indices, table_flat)


def kernel(indices, table):
    out_t = _lookup(indices, table.T.reshape(_V * _D))
    return out_t.transpose(2, 0, 1)


# speedup vs baseline: 17.0914x; 2.5470x over previous
import jax
import jax.numpy as jnp
from jax import lax
from jax.experimental import pallas as pl
from jax.experimental.pallas import tpu as pltpu
from jax.experimental.pallas import tpu_sc as plsc

_NUM_CORES = 2
_NUM_SUBCORES = 16
_NW = _NUM_CORES * _NUM_SUBCORES
_B, _V = 4096, 102
_D = 64
_BPW = _B // _NW
_L = 16
_NJB = _BPW // _L


def _sc_body(idx_hbm, table_hbm, out_hbm, idx_v, table_v, bufs, ssems):
    wid = lax.axis_index("s") * _NUM_CORES + lax.axis_index("c")
    bcol = wid * _BPW

    pltpu.sync_copy(idx_hbm.at[:, pl.ds(bcol, _BPW)], idx_v)
    pltpu.sync_copy(table_hbm, table_v)

    iota = lax.iota(jnp.int32, _L)

    def compute_block(v, buf):
        for jb in range(_NJB):
            lanes = jb * _L + iota
            iv = idx_v[v, pl.ds(jb * _L, _L)]

            @plsc.parallel_loop(0, _D, step=1, unroll=16)
            def dloop(d):
                col = plsc.load_gather(table_v, [iv + d * _V])
                buf[d, pl.ds(jb * _L, _L)] = col

    def out_slice(v):
        return out_hbm.at[v, :, pl.ds(bcol, _BPW)]

    def store(v, b):
        pltpu.async_copy(bufs[b], out_slice(v), ssems[b])

    def wait_store(v, b):
        pltpu.make_async_copy(bufs[b], out_slice(v), ssems[b]).wait()

    for b in (0, 1):
        compute_block(jnp.int32(b), bufs[b])
        store(b, b)

    def body(p, carry):
        for b in (0, 1):
            v = p * 2 + b
            wait_store(v - 2, b)
            compute_block(v, bufs[b])
            store(v, b)
        return carry

    lax.fori_loop(1, _V // 2, body, 0)

    wait_store(_V - 2, 0)
    wait_store(_V - 1, 1)


@jax.jit
def _lookup(indices, table_flat):
    mesh = plsc.VectorSubcoreMesh(core_axis_name="c", subcore_axis_name="s")
    f = pl.kernel(
        _sc_body,
        out_type=jax.ShapeDtypeStruct((_V, _D, _B), jnp.float32),
        mesh=mesh,
        scratch_types=[
            pltpu.VMEM((_V, _BPW), jnp.int32),
            pltpu.VMEM((_V * _D,), jnp.float32),
            [pltpu.VMEM((_D, _BPW), jnp.float32) for _ in range(2)],
            [pltpu.SemaphoreType.DMA for _ in range(2)],
        ],
        compiler_params=pltpu.CompilerParams(
            use_tc_tiling_on_sc=True, needs_layout_passes=False
        ),
    )
    return f(indices, table_flat)


def kernel(indices, table):
    out_t = _lookup(indices.T, table.T.reshape(_V * _D))
    return out_t.transpose(2, 0, 1)
